# Initial kernel scaffold; baseline (speedup 1.0000x reference)
import jax
import jax.numpy as jnp
from jax.experimental import pallas as pl


def kernel(input_ids, mask_prob, keep_replace_prob, standard_tokens, special_tokens):
    raise NotImplementedError("write your pallas kernel here")


# speedup vs baseline: 137.0679x; 137.0679x over previous
import numpy as np
import jax
import jax.numpy as jnp
from jax import lax
from jax.experimental import pallas as pl
from jax.experimental.pallas import tpu as pltpu

_BATCH, _SEQ = 128, 8192
_ROWS_PER_BLOCK = 8
_MASK_TOKEN_ID = 4

_ROTS = ((13, 15, 26, 6), (17, 29, 16, 24))


def _np_threefry2x32(ks0, ks1, x0, x1):
    ks0 = np.uint32(ks0)
    ks1 = np.uint32(ks1)
    ks2 = np.uint32(ks0 ^ ks1 ^ np.uint32(0x1BD11BDA))
    x0 = (np.asarray(x0, np.uint32) + ks0).astype(np.uint32)
    x1 = (np.asarray(x1, np.uint32) + ks1).astype(np.uint32)
    keys = [ks0, ks1, ks2]
    for i in range(5):
        for r in _ROTS[i % 2]:
            x0 = (x0 + x1).astype(np.uint32)
            x1 = ((x1 << np.uint32(r)) | (x1 >> np.uint32(32 - r))).astype(np.uint32)
            x1 = (x1 ^ x0).astype(np.uint32)
        x0 = (x0 + keys[(i + 1) % 3]).astype(np.uint32)
        x1 = (x1 + keys[(i + 2) % 3] + np.uint32(i + 1)).astype(np.uint32)
    return x0, x1


def _derive_subkeys():
    cnt = np.arange(4, dtype=np.uint32)
    a, b = _np_threefry2x32(0, 42, np.zeros(4, np.uint32), cnt)
    k_incl = (int(a[0]), int(b[0]))
    k_mask = (int(a[1]), int(b[1]))
    k_rand = (int(a[2]), int(b[2]))
    k_ids = (int(a[3]), int(b[3]))
    c, d = _np_threefry2x32(k_ids[0], k_ids[1], np.zeros(2, np.uint32), np.arange(2, dtype=np.uint32))
    k_hi = (int(c[0]), int(d[0]))
    k_lo = (int(c[1]), int(d[1]))
    return k_incl, k_mask, k_rand, k_hi, k_lo


_K_INCL, _K_MASK, _K_RAND, _K_HI, _K_LO = _derive_subkeys()


def _tf_bits(key, cnt_u32):
    ks0, ks1 = np.uint32(key[0]), np.uint32(key[1])
    ks2 = np.uint32(ks0 ^ ks1 ^ np.uint32(0x1BD11BDA))
    keys = (int(ks0), int(ks1), int(ks2))
    x0 = jnp.full(cnt_u32.shape, keys[0], jnp.uint32)
    x1 = cnt_u32 + jnp.uint32(keys[1])
    for i in range(5):
        for r in _ROTS[i % 2]:
            x0 = x0 + x1
            x1 = (x1 << jnp.uint32(r)) | (x1 >> jnp.uint32(32 - r))
            x1 = x1 ^ x0
        x0 = x0 + jnp.uint32(keys[(i + 1) % 3])
        x1 = x1 + jnp.uint32((keys[(i + 2) % 3] + i + 1) % (1 << 32))
    return x0 ^ x1


def _u01(bits_u32):
    fb = (bits_u32 >> jnp.uint32(9)) | jnp.uint32(0x3F800000)
    return lax.bitcast_convert_type(fb, jnp.float32) - jnp.float32(1.0)


def _umod(x_u32, span):
    xi = x_u32.astype(jnp.int32)
    xf = xi.astype(jnp.float32)
    xf = jnp.where(xi < 0, xf + jnp.float32(4294967296.0), xf)
    q = (xf * jnp.float32(1.0 / span)).astype(jnp.int32)
    r = xi - q * jnp.int32(span)
    r = jnp.where(r < 0, r + jnp.int32(span), r)
    r = jnp.where(r >= jnp.int32(span), r - jnp.int32(span), r)
    return r


def _mlm_body(span, n_special, ids_ref, mlm_ref, portion_ref, special_ref,
              std0_ref, out_ref, labels_ref):
    g = pl.program_id(0)
    ids = ids_ref[...]
    shape = ids.shape

    row = lax.broadcasted_iota(jnp.uint32, shape, 0)
    col = lax.broadcasted_iota(jnp.uint32, shape, 1)
    cnt = (jnp.uint32(g * _ROWS_PER_BLOCK * _SEQ)
           + row * jnp.uint32(_SEQ) + col)

    mlm = mlm_ref[0]
    portion = portion_ref[0]

    special = ids == special_ref[0]
    for t in range(1, n_special):
        special = special | (ids == special_ref[t])

    u_incl = _u01(_tf_bits(_K_INCL, cnt))
    incl = jnp.logical_and(jnp.logical_not(special), u_incl < mlm)
    labels_ref[...] = jnp.where(incl, ids, jnp.int32(-100))

    u_mask = _u01(_tf_bits(_K_MASK, cnt))
    rep_mask = jnp.logical_and(incl, u_mask < portion)

    u_rand = _u01(_tf_bits(_K_RAND, cnt))
    rep_rand = jnp.logical_and(jnp.logical_and(incl, jnp.logical_not(rep_mask)),
                               u_rand < jnp.float32(0.5))

    hi = _umod(_tf_bits(_K_HI, cnt), span)
    lo = _umod(_tf_bits(_K_LO, cnt), span)
    mult = (2 ** 16 % span) ** 2 % span
    offset = _umod((hi * jnp.int32(mult) + lo).astype(jnp.uint32), span)
    rand_tok = offset + std0_ref[0]

    out = jnp.where(rep_mask, jnp.int32(_MASK_TOKEN_ID), ids)
    out_ref[...] = jnp.where(rep_rand, rand_tok, out)


def kernel(input_ids, mask_prob, keep_replace_prob, standard_tokens, special_tokens):
    mlm = mask_prob + 2.0 * keep_replace_prob
    portion = mask_prob / mlm
    span = int(standard_tokens.shape[0])
    n_special = int(special_tokens.shape[0])

    grid = (_BATCH // _ROWS_PER_BLOCK,)
    blk = (_ROWS_PER_BLOCK, _SEQ)

    out_shape = (
        jax.ShapeDtypeStruct((_BATCH, _SEQ), jnp.int32),
        jax.ShapeDtypeStruct((_BATCH, _SEQ), jnp.int32),
    )
    smem = pltpu.SMEM
    out_ids, labels = pl.pallas_call(
        lambda *refs: _mlm_body(span, n_special, *refs),
        grid=grid,
        in_specs=[
            pl.BlockSpec(blk, lambda g: (g, 0)),
            pl.BlockSpec(memory_space=smem),
            pl.BlockSpec(memory_space=smem),
            pl.BlockSpec(memory_space=smem),
            pl.BlockSpec(memory_space=smem),
        ],
        out_specs=(
            pl.BlockSpec(blk, lambda g: (g, 0)),
            pl.BlockSpec(blk, lambda g: (g, 0)),
        ),
        out_shape=out_shape,
    )(input_ids, mlm, portion, special_tokens, standard_tokens[:1])
    return (out_ids, labels)
